# baseline (speedup 1.0000x reference)
import functools

import jax
import jax.numpy as jnp
from jax import lax
from jax.experimental import pallas as pl
from jax.experimental.pallas import tpu as pltpu
from jax.experimental.pallas import tpu_sc as plsc

B = 16
MAXLEN = 2048
TOTAL = 16384
D = 512
CU_PAD = 32

NC = 2
NS = 16
NW = NC * NS
RPW = (B * MAXLEN) // NW
WPB = MAXLEN // RPW
LOG2_RPW = RPW.bit_length() - 1

_mesh = plsc.VectorSubcoreMesh(core_axis_name="c", subcore_axis_name="s")


def _body(flat_hbm, cu_hbm, zeros_hbm, out_hbm, cu_v, sem):
    wid = lax.axis_index("s") * NC + lax.axis_index("c")
    b = wid // WPB
    p0 = (wid % WPB) * RPW

    pltpu.sync_copy(cu_hbm, cu_v)
    v0 = cu_v[pl.ds(0, 16)]
    v1 = cu_v[pl.ds(16, 16)]
    iota = lax.iota(jnp.int32, 16)

    def _lane(vec, i):
        return jnp.sum(jnp.where(iota == i, vec, 0))

    cu_b = _lane(v0, b)
    cu_b1 = _lane(v0, b + 1) + _lane(v1, b - 15)

    seg_start = cu_b + p0
    valid = jnp.clip(cu_b1 - cu_b - p0, 0, RPW)
    pad = RPW - valid

    def _descs():
        pairs = []
        off = jnp.int32(0)
        for k in range(LOG2_RPW, -1, -1):
            size = 1 << k
            bit = (valid >> k) & 1
            pairs.append(
                (
                    bit,
                    pltpu.make_async_copy(
                        flat_hbm.at[pl.ds(seg_start + off, size)],
                        out_hbm.at[b, pl.ds(p0 + off, size)],
                        sem,
                    ),
                )
            )
            off = off + bit * size
        zoff = valid
        for k in range(LOG2_RPW, -1, -1):
            size = 1 << k
            bit = (pad >> k) & 1
            pairs.append(
                (
                    bit,
                    pltpu.make_async_copy(
                        zeros_hbm.at[pl.ds(0, size)],
                        out_hbm.at[b, pl.ds(p0 + zoff, size)],
                        sem,
                    ),
                )
            )
            zoff = zoff + bit * size
        return pairs

    for bit, desc in _descs():
        pl.when(bit == 1)(desc.start)
    for bit, desc in _descs():
        pl.when(bit == 1)(desc.wait)


_to_dense = functools.partial(
    pl.kernel,
    out_type=jax.ShapeDtypeStruct((B, MAXLEN, D), jnp.float32),
    mesh=_mesh,
    scratch_types=[pltpu.VMEM((CU_PAD,), jnp.int32), pltpu.SemaphoreType.DMA],
    compiler_params=pltpu.CompilerParams(
        use_tc_tiling_on_sc=False, needs_layout_passes=False
    ),
)(_body)


def kernel(flat, cu_seqlens):
    cu = jnp.zeros((CU_PAD,), jnp.int32)
    cu = cu.at[: cu_seqlens.shape[0]].set(cu_seqlens.astype(jnp.int32))
    zeros = jnp.zeros((RPW, D), jnp.float32)
    return _to_dense(flat, cu, zeros)


# speedup vs baseline: 13.0301x; 13.0301x over previous
import functools

import jax
import jax.numpy as jnp
from jax import lax
from jax.experimental import pallas as pl
from jax.experimental.pallas import tpu as pltpu
from jax.experimental.pallas import tpu_sc as plsc

B = 16
MAXLEN = 2048
TOTAL = 16384
D = 512
CU_PAD = 32

NC = 2
NS = 16
NW = NC * NS
RPW = (B * MAXLEN) // NW
WPB = MAXLEN // RPW
C = 64
NCHUNK = RPW // C

_mesh = plsc.VectorSubcoreMesh(core_axis_name="c", subcore_axis_name="s")


def _body(flat_hbm, cu_hbm, zeros_hbm, out_hbm, cu_v, buf0, buf1, zbuf,
          gsem, ssem, zsem):
    wid = lax.axis_index("s") * NC + lax.axis_index("c")
    b = wid // WPB
    p0 = (wid % WPB) * RPW

    pltpu.sync_copy(cu_hbm, cu_v)
    pltpu.sync_copy(zeros_hbm, zbuf)
    v0 = cu_v[pl.ds(0, 16)]
    v1 = cu_v[pl.ds(16, 16)]
    iota = lax.iota(jnp.int32, 16)

    def _lane(vec, i):
        return jnp.sum(jnp.where(iota == i, vec, 0))

    cu_b = _lane(v0, b)
    cu_b1 = _lane(v0, b + 1) + _lane(v1, b - 15)

    seg_start = cu_b + p0
    v = jnp.clip(cu_b1 - cu_b - p0, 0, RPW)

    bufs = [buf0, buf1]

    def full(j):
        return v >= (j + 1) * C

    def zero(j):
        return v <= j * C

    def gather(j):
        return pltpu.make_async_copy(
            flat_hbm.at[pl.ds(seg_start + j * C, C)], bufs[j % 2], gsem)

    def scatter(j):
        return pltpu.make_async_copy(
            bufs[j % 2], out_hbm.at[b, pl.ds(p0 + j * C, C)], ssem)

    def zscatter(j):
        return pltpu.make_async_copy(
            zbuf, out_hbm.at[b, pl.ds(p0 + j * C, C)], zsem)

    for j in range(NCHUNK):
        pl.when(zero(j))(zscatter(j).start)

    pl.when(full(0))(gather(0).start)
    pl.when(full(1))(gather(1).start)
    for j in range(NCHUNK):
        pl.when(full(j))(gather(j).wait)
        pl.when(full(j))(scatter(j).start)
        pl.when(full(j))(scatter(j).wait)
        if j + 2 < NCHUNK:
            pl.when(full(j + 2))(gather(j + 2).start)

    rem = v % C
    base = (v // C) * C

    def piece_copy(off, size):
        def _do():
            g = pltpu.make_async_copy(
                flat_hbm.at[pl.ds(seg_start + base + off, size)],
                buf0.at[pl.ds(0, size)], gsem)
            g.start()
            g.wait()
            s = pltpu.make_async_copy(
                buf0.at[pl.ds(0, size)],
                out_hbm.at[b, pl.ds(p0 + base + off, size)], ssem)
            s.start()
            s.wait()
        return _do

    def piece_zero(off, size):
        return pltpu.make_async_copy(
            zbuf.at[pl.ds(0, size)],
            out_hbm.at[b, pl.ds(p0 + base + off, size)], zsem)

    off = jnp.int32(0)
    kmax = C.bit_length() - 2
    for k in range(kmax, -1, -1):
        size = 1 << k
        bit = (rem >> k) & 1
        pl.when(bit == 1)(piece_copy(off, size))
        off = off + bit * size
    pad = jnp.where(rem > 0, C - rem, 0)
    zoff = rem
    pieces = []
    for k in range(kmax, -1, -1):
        size = 1 << k
        bit = (pad >> k) & 1
        pieces.append((bit, piece_zero(zoff, size)))
        zoff = zoff + bit * size
    for bit, desc in pieces:
        pl.when(bit == 1)(desc.start)

    for bit, desc in pieces:
        pl.when(bit == 1)(desc.wait)
    for j in range(NCHUNK):
        pl.when(zero(j))(zscatter(j).wait)


_to_dense = functools.partial(
    pl.kernel,
    out_type=jax.ShapeDtypeStruct((B, MAXLEN, D), jnp.float32),
    mesh=_mesh,
    scratch_types=[
        pltpu.VMEM((CU_PAD,), jnp.int32),
        pltpu.VMEM((C, D), jnp.float32),
        pltpu.VMEM((C, D), jnp.float32),
        pltpu.VMEM((C, D), jnp.float32),
        pltpu.SemaphoreType.DMA,
        pltpu.SemaphoreType.DMA,
        pltpu.SemaphoreType.DMA,
    ],
    compiler_params=pltpu.CompilerParams(
        use_tc_tiling_on_sc=False, needs_layout_passes=False
    ),
)(_body)


def kernel(flat, cu_seqlens):
    cu = jnp.zeros((CU_PAD,), jnp.int32)
    cu = cu.at[: cu_seqlens.shape[0]].set(cu_seqlens.astype(jnp.int32))
    zeros = jnp.zeros((C, D), jnp.float32)
    return _to_dense(flat, cu, zeros)


# speedup vs baseline: 25.7615x; 1.9771x over previous
import functools

import jax
import jax.numpy as jnp
from jax import lax
from jax.experimental import pallas as pl
from jax.experimental.pallas import tpu as pltpu
from jax.experimental.pallas import tpu_sc as plsc

B = 16
MAXLEN = 2048
TOTAL = 16384
D = 512
CU_PAD = 32

NC = 2
NS = 16
NW = NC * NS
RPW = (B * MAXLEN) // NW
WPB = MAXLEN // RPW
C = 64
NCHUNK = RPW // C

_mesh = plsc.VectorSubcoreMesh(core_axis_name="c", subcore_axis_name="s")


def _body(flat_hbm, cu_hbm, zeros_hbm, out_hbm, cu_v, buf0, buf1, zbuf,
          idx0, idx1, gsem, ssem, zsem):
    wid = lax.axis_index("s") * NC + lax.axis_index("c")
    b = wid // WPB
    row0 = wid * RPW

    pltpu.sync_copy(cu_hbm, cu_v)
    pltpu.sync_copy(zeros_hbm, zbuf)
    v0 = cu_v[pl.ds(0, 16)]
    v1 = cu_v[pl.ds(16, 16)]
    iota = lax.iota(jnp.int32, 16)

    def _lane(vec, i):
        return jnp.sum(jnp.where(iota == i, vec, 0))

    cu_b = _lane(v0, b)
    cu_b1 = _lane(v0, b + 1) + _lane(v1, b - 15)

    p0 = (wid % WPB) * RPW
    seg_start = cu_b + p0
    v = jnp.clip(cu_b1 - cu_b - p0, 0, RPW)

    bufs = [buf0, buf1]
    idxs = [idx0, idx1]

    def full(j):
        return v >= (j + 1) * C

    def zero(j):
        return v <= j * C

    def fill_idx(idx_ref, base_idx):
        for t in range(C // 16):
            idx_ref[pl.ds(16 * t, 16)] = base_idx + 16 * t + iota

    def gather(j):
        return pltpu.make_async_copy(
            flat_hbm.at[idxs[j % 2]], bufs[j % 2], gsem)

    def scatter(j):
        return pltpu.make_async_copy(
            bufs[j % 2],
            out_hbm.at[pl.ds(pl.multiple_of(row0 + j * C, C), C)], ssem)

    def zscatter(j):
        return pltpu.make_async_copy(
            zbuf, out_hbm.at[pl.ds(pl.multiple_of(row0 + j * C, C), C)], zsem)

    for j in range(NCHUNK):
        pl.when(zero(j))(zscatter(j).start)

    def prime(j):
        def _do():
            fill_idx(idxs[j % 2], seg_start + j * C)
            gather(j).start()
        return _do

    pl.when(full(0))(prime(0))
    pl.when(full(1))(prime(1))
    for j in range(NCHUNK):
        pl.when(full(j))(gather(j).wait)
        pl.when(full(j))(scatter(j).start)
        pl.when(full(j))(scatter(j).wait)
        if j + 2 < NCHUNK:
            pl.when(full(j + 2))(prime(j + 2))

    rem = v % C
    base = (v // C) * C
    rem8 = (rem + 7) & ~7

    def _partial():
        for t in range(C // 16):
            idx0[pl.ds(16 * t, 16)] = (
                seg_start + base + jnp.minimum(16 * t + iota, rem - 1))
        g = pltpu.make_async_copy(flat_hbm.at[idx0], buf0, gsem)
        g.start()
        g.wait()

        def _zrow(i, _):
            for cc in range(0, D, 16):
                buf0[i, pl.ds(cc, 16)] = jnp.zeros((16,), jnp.float32)
            return 0

        lax.fori_loop(rem, rem8, _zrow, 0)

        off = jnp.int32(0)
        for k in (6, 5, 4, 3):
            size = 1 << k
            bit = (rem8 >> k) & 1

            def _sc(off=off, size=size):
                sc = pltpu.make_async_copy(
                    buf0.at[pl.ds(pl.multiple_of(off, 8), size)],
                    out_hbm.at[pl.ds(pl.multiple_of(row0 + base + off, 8), size)],
                    ssem)
                sc.start()
                sc.wait()

            pl.when(bit == 1)(_sc)
            off = off + bit * size

        pad = C - rem8
        zoff = rem8
        for k in (5, 4, 3):
            size = 1 << k
            bit = (pad >> k) & 1

            def _zc(zoff=zoff, size=size):
                zc = pltpu.make_async_copy(
                    zbuf.at[pl.ds(0, size)],
                    out_hbm.at[pl.ds(pl.multiple_of(row0 + base + zoff, 8), size)],
                    zsem)
                zc.start()
                zc.wait()

            pl.when(bit == 1)(_zc)
            zoff = zoff + bit * size

    pl.when(rem > 0)(_partial)

    for j in range(NCHUNK):
        pl.when(zero(j))(zscatter(j).wait)


_to_dense = functools.partial(
    pl.kernel,
    out_type=jax.ShapeDtypeStruct((B * MAXLEN, D), jnp.float32),
    mesh=_mesh,
    scratch_types=[
        pltpu.VMEM((CU_PAD,), jnp.int32),
        pltpu.VMEM((C, D), jnp.float32),
        pltpu.VMEM((C, D), jnp.float32),
        pltpu.VMEM((C, D), jnp.float32),
        pltpu.VMEM((C,), jnp.int32),
        pltpu.VMEM((C,), jnp.int32),
        pltpu.SemaphoreType.DMA,
        pltpu.SemaphoreType.DMA,
        pltpu.SemaphoreType.DMA,
    ],
    compiler_params=pltpu.CompilerParams(needs_layout_passes=False),
)(_body)


def kernel(flat, cu_seqlens):
    cu = jnp.zeros((CU_PAD,), jnp.int32)
    cu = cu.at[: cu_seqlens.shape[0]].set(cu_seqlens.astype(jnp.int32))
    zeros = jnp.zeros((C, D), jnp.float32)
    out = _to_dense(flat, cu, zeros)
    return out.reshape(B, MAXLEN, D)


# speedup vs baseline: 27.4343x; 1.0649x over previous
import functools

import jax
import jax.numpy as jnp
from jax import lax
from jax.experimental import pallas as pl
from jax.experimental.pallas import tpu as pltpu
from jax.experimental.pallas import tpu_sc as plsc

B = 16
MAXLEN = 2048
TOTAL = 16384
D = 512
CU_PAD = 32

NC = 2
NS = 16
NW = NC * NS
RPW = (B * MAXLEN) // NW
WPB = MAXLEN // RPW
C = 64
NCHUNK = RPW // C

_mesh = plsc.VectorSubcoreMesh(core_axis_name="c", subcore_axis_name="s")


def _body(flat_hbm, cu_hbm, zeros_hbm, out_hbm, cu_v, buf0, buf1, zbuf,
          idx0, idx1, gsem, ssem, zsem):
    wid = lax.axis_index("s") * NC + lax.axis_index("c")
    b = wid // WPB
    half = wid % WPB
    row0 = b * MAXLEN

    pltpu.sync_copy(cu_hbm, cu_v)
    pltpu.sync_copy(zeros_hbm, zbuf)
    v0 = cu_v[pl.ds(0, 16)]
    v1 = cu_v[pl.ds(16, 16)]
    iota = lax.iota(jnp.int32, 16)

    def _lane(vec, i):
        return jnp.sum(jnp.where(iota == i, vec, 0))

    cu_b = _lane(v0, b)
    cu_b1 = _lane(v0, b + 1) + _lane(v1, b - 15)

    seg_start = cu_b
    v = jnp.clip(cu_b1 - cu_b, 0, MAXLEN)

    bufs = [buf0, buf1]
    idxs = [idx0, idx1]

    def jj(j):
        return WPB * j + half

    def full(j):
        return v >= (jj(j) + 1) * C

    def zero(j):
        return v <= jj(j) * C

    def fill_idx(idx_ref, base_idx):
        for t in range(C // 16):
            idx_ref[pl.ds(16 * t, 16)] = base_idx + 16 * t + iota

    def gather(j):
        return pltpu.make_async_copy(
            flat_hbm.at[idxs[j % 2]], bufs[j % 2], gsem)

    def scatter(j):
        return pltpu.make_async_copy(
            bufs[j % 2],
            out_hbm.at[pl.ds(pl.multiple_of(row0 + jj(j) * C, C), C)], ssem)

    def zscatter(j):
        return pltpu.make_async_copy(
            zbuf,
            out_hbm.at[pl.ds(pl.multiple_of(row0 + jj(j) * C, C), C)], zsem)

    for j in range(NCHUNK):
        pl.when(zero(j))(zscatter(j).start)

    def prime(j):
        def _do():
            fill_idx(idxs[j % 2], seg_start + jj(j) * C)
            gather(j).start()
        return _do

    pl.when(full(0))(prime(0))
    pl.when(full(1))(prime(1))
    for j in range(NCHUNK):
        pl.when(full(j))(gather(j).wait)
        pl.when(full(j))(scatter(j).start)
        pl.when(full(j))(scatter(j).wait)
        if j + 2 < NCHUNK:
            pl.when(full(j + 2))(prime(j + 2))

    rem = v % C
    base = (v // C) * C
    rem8 = (rem + 7) & ~7
    mine = (rem > 0) & ((v // C) % WPB == half)

    def _partial():
        for t in range(C // 16):
            idx0[pl.ds(16 * t, 16)] = (
                seg_start + base + jnp.minimum(16 * t + iota, rem - 1))
        g = pltpu.make_async_copy(flat_hbm.at[idx0], buf0, gsem)
        g.start()
        g.wait()

        def _zrow(i, _):
            for cc in range(0, D, 16):
                buf0[i, pl.ds(cc, 16)] = jnp.zeros((16,), jnp.float32)
            return 0

        lax.fori_loop(rem, rem8, _zrow, 0)

        off = jnp.int32(0)
        for k in (6, 5, 4, 3):
            size = 1 << k
            bit = (rem8 >> k) & 1

            def _sc(off=off, size=size):
                sc = pltpu.make_async_copy(
                    buf0.at[pl.ds(pl.multiple_of(off, 8), size)],
                    out_hbm.at[pl.ds(pl.multiple_of(row0 + base + off, 8), size)],
                    ssem)
                sc.start()
                sc.wait()

            pl.when(bit == 1)(_sc)
            off = off + bit * size

        pad = C - rem8
        zoff = rem8
        for k in (5, 4, 3):
            size = 1 << k
            bit = (pad >> k) & 1

            def _zc(zoff=zoff, size=size):
                zc = pltpu.make_async_copy(
                    zbuf.at[pl.ds(0, size)],
                    out_hbm.at[pl.ds(pl.multiple_of(row0 + base + zoff, 8), size)],
                    zsem)
                zc.start()
                zc.wait()

            pl.when(bit == 1)(_zc)
            zoff = zoff + bit * size

    pl.when(mine)(_partial)

    for j in range(NCHUNK):
        pl.when(zero(j))(zscatter(j).wait)


_to_dense = functools.partial(
    pl.kernel,
    out_type=jax.ShapeDtypeStruct((B * MAXLEN, D), jnp.float32),
    mesh=_mesh,
    scratch_types=[
        pltpu.VMEM((CU_PAD,), jnp.int32),
        pltpu.VMEM((C, D), jnp.float32),
        pltpu.VMEM((C, D), jnp.float32),
        pltpu.VMEM((C, D), jnp.float32),
        pltpu.VMEM((C,), jnp.int32),
        pltpu.VMEM((C,), jnp.int32),
        pltpu.SemaphoreType.DMA,
        pltpu.SemaphoreType.DMA,
        pltpu.SemaphoreType.DMA,
    ],
    compiler_params=pltpu.CompilerParams(needs_layout_passes=False),
)(_body)


def kernel(flat, cu_seqlens):
    cu = jnp.zeros((CU_PAD,), jnp.int32)
    cu = cu.at[: cu_seqlens.shape[0]].set(cu_seqlens.astype(jnp.int32))
    zeros = jnp.zeros((C, D), jnp.float32)
    out = _to_dense(flat, cu, zeros)
    return out.reshape(B, MAXLEN, D)


# speedup vs baseline: 28.4658x; 1.0376x over previous
import functools

import jax
import jax.numpy as jnp
from jax import lax
from jax.experimental import pallas as pl
from jax.experimental.pallas import tpu as pltpu
from jax.experimental.pallas import tpu_sc as plsc

B = 16
MAXLEN = 2048
TOTAL = 16384
D = 512
CU_PAD = 32

NC = 2
NS = 16
NW = NC * NS
RPW = (B * MAXLEN) // NW
WPB = MAXLEN // RPW
C = 64
NCHUNK = RPW // C
NBUF = 3
ZR = 32

_mesh = plsc.VectorSubcoreMesh(core_axis_name="c", subcore_axis_name="s")


def _body(flat_hbm, cu_hbm, zeros_hbm, out_hbm, cu_v, buf0, buf1, buf2,
          zbuf, idx0, idx1, idx2, gsem, ssem, zsem):
    wid = lax.axis_index("s") * NC + lax.axis_index("c")
    b = wid // WPB
    half = wid % WPB
    row0 = b * MAXLEN

    pltpu.sync_copy(cu_hbm, cu_v)
    pltpu.sync_copy(zeros_hbm, zbuf)
    v0 = cu_v[pl.ds(0, 16)]
    v1 = cu_v[pl.ds(16, 16)]
    iota = lax.iota(jnp.int32, 16)

    def _lane(vec, i):
        return jnp.sum(jnp.where(iota == i, vec, 0))

    cu_b = _lane(v0, b)
    cu_b1 = _lane(v0, b + 1) + _lane(v1, b - 15)

    seg_start = cu_b
    v = jnp.clip(cu_b1 - cu_b, 0, MAXLEN)

    bufs = [buf0, buf1, buf2]
    idxs = [idx0, idx1, idx2]

    def jj(j):
        return WPB * j + half

    def full(j):
        return v >= (jj(j) + 1) * C

    def zero(j):
        return v <= jj(j) * C

    def fill_idx(idx_ref, base_idx):
        for t in range(C // 16):
            idx_ref[pl.ds(16 * t, 16)] = base_idx + 16 * t + iota

    def gather(j):
        return pltpu.make_async_copy(
            flat_hbm.at[idxs[j % NBUF]], bufs[j % NBUF], gsem)

    def scatter(j):
        return pltpu.make_async_copy(
            bufs[j % NBUF],
            out_hbm.at[pl.ds(pl.multiple_of(row0 + jj(j) * C, C), C)], ssem)

    def zscatter(j, h):
        return pltpu.make_async_copy(
            zbuf,
            out_hbm.at[
                pl.ds(pl.multiple_of(row0 + jj(j) * C + h * ZR, ZR), ZR)],
            zsem)

    for j in range(NCHUNK):
        for h in range(C // ZR):
            pl.when(zero(j))(zscatter(j, h).start)

    def prime(j):
        def _do():
            fill_idx(idxs[j % NBUF], seg_start + jj(j) * C)
            gather(j).start()
        return _do

    for j in range(NBUF - 1):
        pl.when(full(j))(prime(j))
    for j in range(NCHUNK):
        pl.when(full(j))(gather(j).wait)
        pl.when(full(j))(scatter(j).start)
        if j >= 1:
            pl.when(full(j - 1))(scatter(j - 1).wait)
        if j + NBUF - 1 < NCHUNK:
            pl.when(full(j + NBUF - 1))(prime(j + NBUF - 1))
    pl.when(full(NCHUNK - 1))(scatter(NCHUNK - 1).wait)

    rem = v % C
    base = (v // C) * C
    rem8 = (rem + 7) & ~7
    mine = (rem > 0) & ((v // C) % WPB == half)

    def _partial():
        for t in range(C // 16):
            idx0[pl.ds(16 * t, 16)] = (
                seg_start + base + jnp.minimum(16 * t + iota, rem - 1))
        g = pltpu.make_async_copy(flat_hbm.at[idx0], buf0, gsem)
        g.start()
        g.wait()

        def _zrow(i, _):
            for cc in range(0, D, 16):
                buf0[i, pl.ds(cc, 16)] = jnp.zeros((16,), jnp.float32)
            return 0

        lax.fori_loop(rem, rem8, _zrow, 0)

        off = jnp.int32(0)
        for k in (6, 5, 4, 3):
            size = 1 << k
            bit = (rem8 >> k) & 1

            def _sc(off=off, size=size):
                sc = pltpu.make_async_copy(
                    buf0.at[pl.ds(pl.multiple_of(off, 8), size)],
                    out_hbm.at[pl.ds(pl.multiple_of(row0 + base + off, 8), size)],
                    ssem)
                sc.start()
                sc.wait()

            pl.when(bit == 1)(_sc)
            off = off + bit * size

        pad = C - rem8
        zoff = rem8
        for k in (5, 4, 3):
            size = 1 << k
            bit = (pad >> k) & 1

            def _zc(zoff=zoff, size=size):
                zc = pltpu.make_async_copy(
                    zbuf.at[pl.ds(0, size)],
                    out_hbm.at[pl.ds(pl.multiple_of(row0 + base + zoff, 8), size)],
                    zsem)
                zc.start()
                zc.wait()

            pl.when(bit == 1)(_zc)
            zoff = zoff + bit * size

    pl.when(mine)(_partial)

    for j in range(NCHUNK):
        for h in range(C // ZR):
            pl.when(zero(j))(zscatter(j, h).wait)


_to_dense = functools.partial(
    pl.kernel,
    out_type=jax.ShapeDtypeStruct((B * MAXLEN, D), jnp.float32),
    mesh=_mesh,
    scratch_types=[
        pltpu.VMEM((CU_PAD,), jnp.int32),
        pltpu.VMEM((C, D), jnp.float32),
        pltpu.VMEM((C, D), jnp.float32),
        pltpu.VMEM((C, D), jnp.float32),
        pltpu.VMEM((ZR, D), jnp.float32),
        pltpu.VMEM((C,), jnp.int32),
        pltpu.VMEM((C,), jnp.int32),
        pltpu.VMEM((C,), jnp.int32),
        pltpu.SemaphoreType.DMA,
        pltpu.SemaphoreType.DMA,
        pltpu.SemaphoreType.DMA,
    ],
    compiler_params=pltpu.CompilerParams(needs_layout_passes=False),
)(_body)


def kernel(flat, cu_seqlens):
    cu = jnp.zeros((CU_PAD,), jnp.int32)
    cu = cu.at[: cu_seqlens.shape[0]].set(cu_seqlens.astype(jnp.int32))
    zeros = jnp.zeros((ZR, D), jnp.float32)
    out = _to_dense(flat, cu, zeros)
    return out.reshape(B, MAXLEN, D)


# speedup vs baseline: 29.5678x; 1.0387x over previous
import functools

import jax
import jax.numpy as jnp
from jax import lax
from jax.experimental import pallas as pl
from jax.experimental.pallas import tpu as pltpu
from jax.experimental.pallas import tpu_sc as plsc

B = 16
MAXLEN = 2048
TOTAL = 16384
D = 512
CU_PAD = 32

NC = 2
NS = 16
NW = NC * NS
RPW = (B * MAXLEN) // NW
WPB = MAXLEN // RPW
C = 64
NCHUNK = RPW // C
NBUF = 3
ZR = 32

_mesh = plsc.VectorSubcoreMesh(core_axis_name="c", subcore_axis_name="s")


def _body(flat_hbm, cu_hbm, zeros_hbm, out_hbm, cu_v, buf0, buf1, buf2,
          zbuf, idx0, idx1, idx2, gsem, ssem, zsem):
    wid = lax.axis_index("s") * NC + lax.axis_index("c")
    b = wid // WPB
    half = wid % WPB
    row0 = b * MAXLEN

    cu_copy = pltpu.make_async_copy(cu_hbm, cu_v, gsem)
    cu_copy.start()
    z_copy = pltpu.make_async_copy(zeros_hbm, zbuf, zsem)
    z_copy.start()
    cu_copy.wait()
    v0 = cu_v[pl.ds(0, 16)]
    v1 = cu_v[pl.ds(16, 16)]
    iota = lax.iota(jnp.int32, 16)

    def _lane(vec, i):
        return jnp.sum(jnp.where(iota == i, vec, 0))

    cu_b = _lane(v0, b)
    cu_b1 = _lane(v0, b + 1) + _lane(v1, b - 15)

    seg_start = cu_b
    v = jnp.clip(cu_b1 - cu_b, 0, MAXLEN)

    bufs = [buf0, buf1, buf2]
    idxs = [idx0, idx1, idx2]

    def jj(j):
        return WPB * j + half

    def full(j):
        return v >= (jj(j) + 1) * C

    def zero(j):
        return v <= jj(j) * C

    def fill_idx(idx_ref, base_idx):
        for t in range(C // 16):
            idx_ref[pl.ds(16 * t, 16)] = base_idx + 16 * t + iota

    def gather(j):
        return pltpu.make_async_copy(
            flat_hbm.at[idxs[j % NBUF]], bufs[j % NBUF], gsem)

    def scatter(j):
        return pltpu.make_async_copy(
            bufs[j % NBUF],
            out_hbm.at[pl.ds(pl.multiple_of(row0 + jj(j) * C, C), C)], ssem)

    def zscatter(j, h):
        return pltpu.make_async_copy(
            zbuf,
            out_hbm.at[
                pl.ds(pl.multiple_of(row0 + jj(j) * C + h * ZR, ZR), ZR)],
            zsem)

    def prime(j):
        def _do():
            fill_idx(idxs[j % NBUF], seg_start + jj(j) * C)
            gather(j).start()
        return _do

    for j in range(NBUF - 1):
        pl.when(full(j))(prime(j))

    z_copy.wait()
    for j in range(NCHUNK):
        for h in range(C // ZR):
            pl.when(zero(j))(zscatter(j, h).start)
    for j in range(NCHUNK):
        pl.when(full(j))(gather(j).wait)
        pl.when(full(j))(scatter(j).start)
        if j >= 1:
            pl.when(full(j - 1))(scatter(j - 1).wait)
        if j + NBUF - 1 < NCHUNK:
            pl.when(full(j + NBUF - 1))(prime(j + NBUF - 1))
    pl.when(full(NCHUNK - 1))(scatter(NCHUNK - 1).wait)

    rem = v % C
    base = (v // C) * C
    rem8 = (rem + 7) & ~7
    mine = (rem > 0) & ((v // C) % WPB == half)

    def _partial():
        for t in range(C // 16):
            idx0[pl.ds(16 * t, 16)] = (
                seg_start + base + jnp.minimum(16 * t + iota, rem - 1))
        g = pltpu.make_async_copy(flat_hbm.at[idx0], buf0, gsem)
        g.start()
        g.wait()

        def _zrow(i, _):
            for cc in range(0, D, 16):
                buf0[i, pl.ds(cc, 16)] = jnp.zeros((16,), jnp.float32)
            return 0

        lax.fori_loop(rem, rem8, _zrow, 0)

        off = jnp.int32(0)
        for k in (6, 5, 4, 3):
            size = 1 << k
            bit = (rem8 >> k) & 1

            def _sc(off=off, size=size):
                sc = pltpu.make_async_copy(
                    buf0.at[pl.ds(pl.multiple_of(off, 8), size)],
                    out_hbm.at[pl.ds(pl.multiple_of(row0 + base + off, 8), size)],
                    ssem)
                sc.start()
                sc.wait()

            pl.when(bit == 1)(_sc)
            off = off + bit * size

        pad = C - rem8
        zoff = rem8
        for k in (5, 4, 3):
            size = 1 << k
            bit = (pad >> k) & 1

            def _zc(zoff=zoff, size=size):
                zc = pltpu.make_async_copy(
                    zbuf.at[pl.ds(0, size)],
                    out_hbm.at[pl.ds(pl.multiple_of(row0 + base + zoff, 8), size)],
                    zsem)
                zc.start()
                zc.wait()

            pl.when(bit == 1)(_zc)
            zoff = zoff + bit * size

    pl.when(mine)(_partial)

    for j in range(NCHUNK):
        for h in range(C // ZR):
            pl.when(zero(j))(zscatter(j, h).wait)


_to_dense = functools.partial(
    pl.kernel,
    out_type=jax.ShapeDtypeStruct((B * MAXLEN, D), jnp.float32),
    mesh=_mesh,
    scratch_types=[
        pltpu.VMEM((CU_PAD,), jnp.int32),
        pltpu.VMEM((C, D), jnp.float32),
        pltpu.VMEM((C, D), jnp.float32),
        pltpu.VMEM((C, D), jnp.float32),
        pltpu.VMEM((ZR, D), jnp.float32),
        pltpu.VMEM((C,), jnp.int32),
        pltpu.VMEM((C,), jnp.int32),
        pltpu.VMEM((C,), jnp.int32),
        pltpu.SemaphoreType.DMA,
        pltpu.SemaphoreType.DMA,
        pltpu.SemaphoreType.DMA,
    ],
    compiler_params=pltpu.CompilerParams(needs_layout_passes=False),
)(_body)


def kernel(flat, cu_seqlens):
    cu = jnp.zeros((CU_PAD,), jnp.int32)
    cu = cu.at[: cu_seqlens.shape[0]].set(cu_seqlens.astype(jnp.int32))
    zeros = jnp.zeros((ZR, D), jnp.float32)
    out = _to_dense(flat, cu, zeros)
    return out.reshape(B, MAXLEN, D)
